# Initial kernel scaffold; baseline (speedup 1.0000x reference)
import jax
import jax.numpy as jnp
from jax.experimental import pallas as pl


def kernel(atom_feats, bond_attr, triplet_feats, h_periodic_complete, edge_index, angle_index, tuple_edge_index, params):
    raise NotImplementedError("write your pallas kernel here")


# speedup vs baseline: 1.2033x; 1.2033x over previous
import functools

import jax
import jax.numpy as jnp
from jax import lax
from jax.experimental import pallas as pl
from jax.experimental.pallas import tpu as pltpu


def _mm_body(x_ref, w_ref, b_ref, o_ref):
    o_ref[...] = (
        jnp.dot(x_ref[...], w_ref[...], preferred_element_type=jnp.float32)
        + b_ref[...]
    )


def _mm(x, w, b, br=1000):
    r, d = x.shape
    k = w.shape[1]
    assert r % br == 0
    return pl.pallas_call(
        _mm_body,
        grid=(r // br,),
        in_specs=[
            pl.BlockSpec((br, d), lambda i: (i, 0)),
            pl.BlockSpec((d, k), lambda i: (0, 0)),
            pl.BlockSpec((1, k), lambda i: (0, 0)),
        ],
        out_specs=pl.BlockSpec((br, k), lambda i: (i, 0)),
        out_shape=jax.ShapeDtypeStruct((r, k), jnp.float32),
    )(x, w, b.reshape(1, k))


def _tables_body(x_ref, wgu_ref, bgu_ref, wgd_ref, bgd_ref, whd_ref, bhd_ref,
                 gu_ref, gd_ref, hd_ref):
    x = x_ref[...]
    gu_ref[...] = jnp.dot(x, wgu_ref[...], preferred_element_type=jnp.float32) + bgu_ref[...]
    gd_ref[...] = jnp.dot(x, wgd_ref[...], preferred_element_type=jnp.float32) + bgd_ref[...]
    hd_ref[...] = jnp.dot(x, whd_ref[...], preferred_element_type=jnp.float32) + bhd_ref[...]


def _node_tables(x, p, br=1000):
    r, d = x.shape
    assert r % br == 0
    wgu = jnp.concatenate([p['Wsg'], p['Wsu']], axis=1)
    bgu = jnp.concatenate([p['bsg'], p['bsu']]).reshape(1, 2 * d)
    return pl.pallas_call(
        _tables_body,
        grid=(r // br,),
        in_specs=[
            pl.BlockSpec((br, d), lambda i: (i, 0)),
            pl.BlockSpec((d, 2 * d), lambda i: (0, 0)),
            pl.BlockSpec((1, 2 * d), lambda i: (0, 0)),
            pl.BlockSpec((d, d), lambda i: (0, 0)),
            pl.BlockSpec((1, d), lambda i: (0, 0)),
            pl.BlockSpec((d, d), lambda i: (0, 0)),
            pl.BlockSpec((1, d), lambda i: (0, 0)),
        ],
        out_specs=[
            pl.BlockSpec((br, 2 * d), lambda i: (i, 0)),
            pl.BlockSpec((br, d), lambda i: (i, 0)),
            pl.BlockSpec((br, d), lambda i: (i, 0)),
        ],
        out_shape=[
            jax.ShapeDtypeStruct((r, 2 * d), jnp.float32),
            jax.ShapeDtypeStruct((r, d), jnp.float32),
            jax.ShapeDtypeStruct((r, d), jnp.float32),
        ],
    )(x, wgu, bgu, p['Wdg'], p['bdg'].reshape(1, d), p['Wdu'], p['bdu'].reshape(1, d))


def _ln(x, g=None, b=None, eps=1e-5):
    m = jnp.mean(x, axis=-1, keepdims=True)
    v = jnp.mean((x - m) ** 2, axis=-1, keepdims=True)
    y = (x - m) * lax.rsqrt(v + eps)
    if g is not None:
        y = y * g + b
    return y


def _silu(x):
    return x * jax.nn.sigmoid(x)


def _edge_math_body(gu_ref, gd_ref, t_ref, ef_ref, g_ref, b_ref,
                    ms_ref, eo_ref):
    d = gd_ref.shape[-1]
    gu = gu_ref[...]
    e_pre = gu[:, :d] + gd_ref[...] + t_ref[...]
    sig = jax.nn.sigmoid(e_pre)
    ms_ref[...] = jnp.concatenate([sig * gu[:, d:], sig], axis=-1)
    eo_ref[...] = ef_ref[...] + _silu(_ln(e_pre, g_ref[...], b_ref[...]))


def _edge_math_tc(gu_g, gd_g, t, ef, g, b, br=1000):
    r, d = t.shape
    return pl.pallas_call(
        _edge_math_body,
        grid=(r // br,),
        in_specs=[
            pl.BlockSpec((br, 2 * d), lambda i: (i, 0)),
            pl.BlockSpec((br, d), lambda i: (i, 0)),
            pl.BlockSpec((br, d), lambda i: (i, 0)),
            pl.BlockSpec((br, d), lambda i: (i, 0)),
            pl.BlockSpec((1, d), lambda i: (0, 0)),
            pl.BlockSpec((1, d), lambda i: (0, 0)),
        ],
        out_specs=[
            pl.BlockSpec((br, 2 * d), lambda i: (i, 0)),
            pl.BlockSpec((br, d), lambda i: (i, 0)),
        ],
        out_shape=[
            jax.ShapeDtypeStruct((r, 2 * d), jnp.float32),
            jax.ShapeDtypeStruct((r, d), jnp.float32),
        ],
    )(gu_g, gd_g, t, ef, g.reshape(1, d), b.reshape(1, d))


def _node_upd_body(hd_ref, ad_ref, node_ref, g_ref, b_ref, o_ref):
    d = hd_ref.shape[-1]
    ad = ad_ref[...]
    h = hd_ref[...] + ad[:, :d] / (ad[:, d:] + 1e-6)
    o_ref[...] = node_ref[...] + _silu(_ln(h, g_ref[...], b_ref[...]))


def _node_upd_t2_body(hd_ref, ad_ref, node_ref, g_ref, b_ref, w_ref, be_ref,
                      bl_ref, t2_ref):
    d = hd_ref.shape[-1]
    ad = ad_ref[...]
    h = hd_ref[...] + ad[:, :d] / (ad[:, d:] + 1e-6)
    bl = node_ref[...] + _silu(_ln(h, g_ref[...], b_ref[...]))
    bl_ref[...] = bl
    t2_ref[...] = jnp.dot(bl, w_ref[...], preferred_element_type=jnp.float32) + be_ref[...]


def _node_upd_t2(hd, ad, node, g, b, weg, beg, br=1000):
    r, d = hd.shape
    return pl.pallas_call(
        _node_upd_t2_body,
        grid=(r // br,),
        in_specs=[
            pl.BlockSpec((br, d), lambda i: (i, 0)),
            pl.BlockSpec((br, 2 * d), lambda i: (i, 0)),
            pl.BlockSpec((br, d), lambda i: (i, 0)),
            pl.BlockSpec((1, d), lambda i: (0, 0)),
            pl.BlockSpec((1, d), lambda i: (0, 0)),
            pl.BlockSpec((d, d), lambda i: (0, 0)),
            pl.BlockSpec((1, d), lambda i: (0, 0)),
        ],
        out_specs=[
            pl.BlockSpec((br, d), lambda i: (i, 0)),
            pl.BlockSpec((br, d), lambda i: (i, 0)),
        ],
        out_shape=[
            jax.ShapeDtypeStruct((r, d), jnp.float32),
            jax.ShapeDtypeStruct((r, d), jnp.float32),
        ],
    )(hd, ad, node, g.reshape(1, d), b.reshape(1, d), weg, beg.reshape(1, d))


def _fusion_body(atom_ref, hd_ref, ad2_ref, ad3_ref, g_ref, b_ref,
                 wf1_ref, bf1_ref, g1_ref, b1_ref, wf2_ref, bf2_ref, o_ref):
    d = hd_ref.shape[-1]
    atom = atom_ref[...]
    hd = hd_ref[...]
    g, b = g_ref[...], b_ref[...]

    ad2 = ad2_ref[...]
    h2 = hd + ad2[:, :d] / (ad2[:, d:] + 1e-6)
    g2l = _ln(atom + _silu(_ln(h2, g, b)))

    ad3 = ad3_ref[...]
    h3 = hd + ad3[:, :d] / (ad3[:, d:] + 1e-6)
    glob = _ln(atom + _silu(_ln(h3, g, b)))

    gate = jnp.concatenate([g2l, glob], axis=-1)
    h = jnp.dot(gate, wf1_ref[...], preferred_element_type=jnp.float32) + bf1_ref[...]
    h = jax.nn.relu(_ln(h, g1_ref[...], b1_ref[...]))
    z = jax.nn.sigmoid(
        jnp.dot(h, wf2_ref[...], preferred_element_type=jnp.float32) + bf2_ref[...])
    o_ref[...] = z * g2l + (1.0 - z) * glob


def _fusion(atom, hd, ad2, ad3, g, b, f, br=1000):
    r, d = atom.shape
    return pl.pallas_call(
        _fusion_body,
        grid=(r // br,),
        in_specs=[
            pl.BlockSpec((br, d), lambda i: (i, 0)),
            pl.BlockSpec((br, d), lambda i: (i, 0)),
            pl.BlockSpec((br, 2 * d), lambda i: (i, 0)),
            pl.BlockSpec((br, 2 * d), lambda i: (i, 0)),
            pl.BlockSpec((1, d), lambda i: (0, 0)),
            pl.BlockSpec((1, d), lambda i: (0, 0)),
            pl.BlockSpec((2 * d, d), lambda i: (0, 0)),
            pl.BlockSpec((1, d), lambda i: (0, 0)),
            pl.BlockSpec((1, d), lambda i: (0, 0)),
            pl.BlockSpec((1, d), lambda i: (0, 0)),
            pl.BlockSpec((d, d), lambda i: (0, 0)),
            pl.BlockSpec((1, d), lambda i: (0, 0)),
        ],
        out_specs=pl.BlockSpec((br, d), lambda i: (i, 0)),
        out_shape=jax.ShapeDtypeStruct((r, d), jnp.float32),
    )(atom, hd, ad2, ad3, g.reshape(1, d), b.reshape(1, d),
      f['Wf1'], f['bf1'].reshape(1, d), f['g1'].reshape(1, d),
      f['b1'].reshape(1, d), f['Wf2'], f['bf2'].reshape(1, d))


def _edge_phase(gu, gd, t, ef, src, dst, g, b):
    gu_g = jnp.take(gu, src, axis=0)
    gd_g = jnp.take(gd, dst, axis=0)
    return _edge_math_tc(gu_g, gd_g, t, ef, g, b)


def _scatter_phase(ms, dst, n):
    return jax.ops.segment_sum(ms, dst, num_segments=n)


def kernel(atom_feats, bond_attr, triplet_feats, h_periodic_complete,
           edge_index, angle_index, tuple_edge_index, params):
    pa, pb, pf = params['angle'], params['atom'], params['fuse']
    n = atom_feats.shape[0]
    e = bond_attr.shape[0]

    gu1, gd1, hd1 = _node_tables(bond_attr, pa)
    t1 = _mm(triplet_feats, pa['Weg'], pa['beg'])
    ms1, triplet_upd = _edge_phase(gu1, gd1, t1, triplet_feats,
                                   angle_index[0], angle_index[1],
                                   pa['ln_e_g'], pa['ln_e_b'])
    ad1 = _scatter_phase(ms1, angle_index[1], e)
    bond_local, t2 = _node_upd_t2(hd1, ad1, bond_attr,
                                  pa['ln_n_g'], pa['ln_n_b'],
                                  pb['Weg'], pb['beg'])

    gua, gda, hda = _node_tables(atom_feats, pb)
    t3 = _mm(h_periodic_complete, pb['Weg'], pb['beg'])

    ms2, bond_upd = _edge_phase(gua, gda, t2, bond_local,
                                edge_index[0], edge_index[1],
                                pb['ln_e_g'], pb['ln_e_b'])
    ad2 = _scatter_phase(ms2, edge_index[1], n)

    ms3, _ = _edge_phase(gua, gda, t3, h_periodic_complete,
                         tuple_edge_index[0], tuple_edge_index[1],
                         pb['ln_e_g'], pb['ln_e_b'])
    ad3 = _scatter_phase(ms3, tuple_edge_index[1], n)

    final = _fusion(atom_feats, hda, ad2, ad3,
                    pb['ln_n_g'], pb['ln_n_b'], pf)
    return (final, bond_upd, triplet_upd)
